# baseline (speedup 1.0000x reference)
import functools

import jax
import jax.numpy as jnp
from jax import lax
from jax.experimental import pallas as pl
from jax.experimental.pallas import tpu as pltpu
from jax.experimental.pallas import tpu_sc as plsc

_N = 320000
_D = 128
_S = 10000
_NC = 2
_NS = 16
_LANES = 16
_NW = _NC * _NS

_CHUNK1 = 400
_CHUNK2 = 128

_EPS = 1e-12


def _make_mesh():
  return plsc.VectorSubcoreMesh(
      core_axis_name="c", subcore_axis_name="s",
      num_cores=_NC, num_subcores=_NS)


def _build_phase1(n, d, s_count, interpret=False):
  nv = d // _LANES
  p = n // _NW
  gpc = _CHUNK1 // 16
  nchunks = p // _CHUNK1
  gpw = p // 16
  total_chunks_all = n // _CHUNK1

  fb = 48
  fb_fire = 32

  def body(t_hbm, seg16_hbm, segp_hbm, stats_hbm, tbuf, segv, fsegv, pgbuf,
           flushbuf, flushidx, accbuf, smem, csem0, csem1, bsem0, bsem1,
           ext_sem):
    wid = lax.axis_index("s") * _NC + lax.axis_index("c")
    g0 = wid * gpw
    csems = (csem0, csem1)
    bsems = (bsem0, bsem1)
    lane0 = jnp.arange(16, dtype=jnp.int32) == 0
    pow2 = jnp.left_shift(jnp.int32(1), jnp.arange(16, dtype=jnp.int32))
    dummy_row = jnp.full((16,), jnp.int32(s_count))

    def reset_idx(sp):
      for i in range(fb // 16):
        flushidx[sp, pl.ds(i * 16, 16)] = dummy_row

    def flush(seg_splat, mx, mn):
      fc = smem[3]
      par = smem[4]
      for v in range(nv):
        mid = (mx[v] + mn[v]) * 0.5
        ldv = (mx[v] - mn[v]) * 0.5
        inv = 1.0 / jnp.maximum(ldv, _EPS)
        flushbuf[par, fc, pl.ds(_LANES * v, _LANES)] = mid
        flushbuf[par, fc, pl.ds(d + _LANES * v, _LANES)] = inv
      plsc.store_scatter(flushidx.at[par], [jnp.full((16,), fc)],
                         seg_splat, mask=lane0)
      smem[3] = fc + 1

    def fire_batch():
      par = smem[4]
      bfired = smem[5]
      for sp in range(2):
        @pl.when(par == sp)
        def _():
          pltpu.async_copy(
              flushbuf.at[sp], stats_hbm.at[flushidx.at[sp]], bsems[sp])

          @pl.when(bfired >= 1)
          def _():
            pltpu.make_async_copy(
                flushbuf.at[1 - sp], stats_hbm.at[flushidx.at[1 - sp]],
                bsems[1 - sp]).wait()
          reset_idx(1 - sp)

      smem[4] = 1 - par
      smem[3] = jnp.int32(0)
      smem[5] = bfired + 1

    def load_row(buf, slot, j):
      return [buf[slot, j, pl.ds(_LANES * v, _LANES)] for v in range(nv)]

    def load_acc():
      return ([accbuf[0, pl.ds(_LANES * v, _LANES)] for v in range(nv)],
              [accbuf[1, pl.ds(_LANES * v, _LANES)] for v in range(nv)])

    def store_acc(mx, mn):
      for v in range(nv):
        accbuf[0, pl.ds(_LANES * v, _LANES)] = mx[v]
        accbuf[1, pl.ds(_LANES * v, _LANES)] = mn[v]

    pltpu.sync_copy(seg16_hbm.at[jnp.maximum(g0 - 1, 0)], pgbuf)
    pgv = pgbuf[...]
    smem[0] = jnp.where(wid == 0, jnp.int32(-1), pgv[15])
    smem[1] = jnp.int32(0)
    smem[3] = jnp.int32(0)
    smem[4] = jnp.int32(0)
    smem[5] = jnp.int32(0)
    reset_idx(0)
    reset_idx(1)

    def issue_chunk(k, s):
      row0 = wid * p + k * _CHUNK1
      pltpu.async_copy(t_hbm.at[pl.ds(row0, _CHUNK1)], tbuf.at[s], csems[s])
      pltpu.async_copy(
          seg16_hbm.at[pl.ds(g0 + k * gpc, gpc)], segv.at[s], csems[s])
      pltpu.async_copy(
          segp_hbm.at[pl.ds(row0, _CHUNK1 + 16)], fsegv.at[s], csems[s])

    def wait_chunk(s):
      pltpu.make_async_copy(
          t_hbm.at[pl.ds(0, _CHUNK1)], tbuf.at[s], csems[s]).wait()
      pltpu.make_async_copy(
          seg16_hbm.at[pl.ds(0, gpc)], segv.at[s], csems[s]).wait()
      pltpu.make_async_copy(
          segp_hbm.at[pl.ds(0, _CHUNK1 + 16)], fsegv.at[s], csems[s]).wait()

    issue_chunk(0, 0)

    def chunk_body(k, c):
      slot = lax.rem(k, 2)

      @pl.when(slot == 0)
      def _():
        wait_chunk(0)

        @pl.when(k + 1 < nchunks)
        def _():
          issue_chunk(k + 1, 1)

      @pl.when(slot == 1)
      def _():
        wait_chunk(1)

        @pl.when(k + 1 < nchunks)
        def _():
          issue_chunk(k + 1, 0)

      def group_body(g, c2):
        prev = smem[0]
        vseg = segv[slot, g]
        last = vseg[15]
        fast_p = last == prev

        @pl.when(fast_p)
        def _():
          mx, mn = load_acc()
          for j in range(16):
            row = load_row(tbuf, slot, g * 16 + j)
            mx = [jnp.maximum(mx[v], row[v]) for v in range(nv)]
            mn = [jnp.minimum(mn[v], row[v]) for v in range(nv)]
          store_acc(mx, mn)

        @pl.when(jnp.logical_not(fast_p))
        def _():
          started0 = smem[1]
          mx, mn = load_acc()
          shifted = fsegv[slot, pl.ds(g * 16 + 7, 16)]
          bvec = vseg != shifted
          mbits = jnp.sum(jnp.where(bvec, pow2, 0))
          prev_splat = jnp.full((16,), prev)

          def slow_row(j, c3):
            started = c3[0]
            mx, mn = list(c3[1:1 + nv]), list(c3[1 + nv:])
            row = load_row(tbuf, slot, g * 16 + j)
            bj = jnp.bitwise_and(jnp.right_shift(mbits, j), 1)
            boundary = bj != 0
            do_flush = jnp.logical_and(boundary, started != 0)

            @pl.when(do_flush)
            def _():
              seg_splat = jnp.where(
                  j == 0, prev_splat,
                  jnp.take_along_axis(
                      vseg,
                      jnp.full((16,), jnp.maximum(j - 1, 0), jnp.int32),
                      axis=0))
              flush(seg_splat, mx, mn)

            mx = [jnp.where(boundary, row[v], jnp.maximum(mx[v], row[v]))
                  for v in range(nv)]
            mn = [jnp.where(boundary, row[v], jnp.minimum(mn[v], row[v]))
                  for v in range(nv)]
            started = jnp.bitwise_or(started, bj)
            return (started,) + tuple(mx) + tuple(mn)

          out = lax.fori_loop(0, 16, slow_row,
                              (started0,) + tuple(mx) + tuple(mn))
          store_acc(list(out[1:1 + nv]), list(out[1 + nv:]))
          smem[0] = last
          smem[1] = out[0]

          @pl.when(smem[3] >= fb_fire)
          def _():
            fire_batch()

        return c2

      return lax.fori_loop(0, gpc, group_body, c)

    lax.fori_loop(0, nchunks, chunk_body, 0)
    prev = smem[0]
    started = smem[1] != 0

    smem[2] = jnp.where(started, jnp.int32(0), jnp.int32(1))

    def ext_chunk(e, c):
      cc = (wid + 1) * nchunks + e

      @pl.when(jnp.logical_and(smem[2] == 0, cc < total_chunks_all))
      def _():
        pltpu.async_copy(
            t_hbm.at[pl.ds(cc * _CHUNK1, _CHUNK1)], tbuf.at[0],
            ext_sem).wait()
        pltpu.async_copy(
            seg16_hbm.at[pl.ds(cc * gpc, gpc)], segv.at[0], ext_sem).wait()

        def ext_group(g, c2):
          @pl.when(smem[2] == 0)
          def _():
            vseg = segv[0, g]
            mx, mn = load_acc()
            mb = jnp.sum(jnp.where(vseg != jnp.full((16,), prev), pow2, 0))

            def ext_row(j, c3):
              mx = list(c3[:nv])
              mn = list(c3[nv:])
              row = load_row(tbuf, 0, g * 16 + j)
              keep = jnp.bitwise_and(
                  mb, jnp.left_shift(jnp.int32(2), j) - 1) == 0
              mx = [jnp.where(keep, jnp.maximum(mx[v], row[v]), mx[v])
                    for v in range(nv)]
              mn = [jnp.where(keep, jnp.minimum(mn[v], row[v]), mn[v])
                    for v in range(nv)]
              return tuple(mx) + tuple(mn)

            out = lax.fori_loop(0, 16, ext_row, tuple(mx) + tuple(mn))
            store_acc(list(out[:nv]), list(out[nv:]))
            smem[2] = jnp.where(mb == 0, jnp.int32(0), jnp.int32(1))

          return c2

        lax.fori_loop(0, gpc, ext_group, 0)

      return c

    lax.fori_loop(0, total_chunks_all - nchunks, ext_chunk, 0)

    @pl.when(started)
    def _():
      mx, mn = load_acc()
      flush(jnp.full((16,), prev), mx, mn)

    @pl.when(smem[3] > 0)
    def _():
      fire_batch()

    @pl.when(smem[5] >= 1)
    def _():
      last = 1 - smem[4]
      for sp in range(2):
        @pl.when(last == sp)
        def _():
          pltpu.make_async_copy(
              flushbuf.at[sp], stats_hbm.at[flushidx.at[sp]],
              bsems[sp]).wait()

  return pl.kernel(
      body,
      out_type=jax.ShapeDtypeStruct((s_count + 8, 2 * d), jnp.float32),
      mesh=_make_mesh(),
      scratch_types=[
          pltpu.VMEM((2, _CHUNK1, d), jnp.float32),
          pltpu.VMEM((2, gpc, 16), jnp.int32),
          pltpu.VMEM((2, _CHUNK1 + 16), jnp.int32),
          pltpu.VMEM((16,), jnp.int32),
          pltpu.VMEM((2, fb, 2 * d), jnp.float32),
          pltpu.VMEM((2, fb), jnp.int32),
          pltpu.VMEM((2, d), jnp.float32),
          pltpu.SMEM((8,), jnp.int32),
          pltpu.SemaphoreType.DMA,
          pltpu.SemaphoreType.DMA,
          pltpu.SemaphoreType.DMA,
          pltpu.SemaphoreType.DMA,
          pltpu.SemaphoreType.DMA,
      ],
      compiler_params=pltpu.CompilerParams(
          use_tc_tiling_on_sc=False, needs_layout_passes=False),
      interpret=interpret,
  )


def _build_phase2(n, d, s_count, interpret=False):
  nv = d // _LANES
  total_chunks = n // _CHUNK2
  iters = -(-total_chunks // _NW)

  def body(t_hbm, segf_hbm, stats_hbm, out_hbm, xbuf, gbuf, idxbuf,
           isem0, isem1, isem2, isem3, gsem0, gsem1, xsem0, xsem1,
           osem0, osem1):
    wid = lax.axis_index("s") * _NC + lax.axis_index("c")
    isems = (isem0, isem1, isem2, isem3)
    gsems = (gsem0, gsem1)
    xsems = (xsem0, xsem1)
    osems = (osem0, osem1)

    def prefetch_idx(k, si):
      cid = wid + k * _NW

      @pl.when(jnp.logical_and(cid >= 0, cid < total_chunks))
      def _():
        pltpu.async_copy(
            segf_hbm.at[pl.ds(cid * _CHUNK2, _CHUNK2)], idxbuf.at[si],
            isems[si])

    def start(k, u):
      s, si = u % 2, u % 4
      cid = wid + k * _NW
      prev_cid = cid - 2 * _NW

      @pl.when(jnp.logical_and(cid >= 0, cid < total_chunks))
      def _():
        pltpu.make_async_copy(
            segf_hbm.at[pl.ds(0, _CHUNK2)], idxbuf.at[si], isems[si]).wait()

        @pl.when(prev_cid >= 0)
        def _():
          pltpu.make_async_copy(
              xbuf.at[s], out_hbm.at[pl.ds(0, _CHUNK2)], osems[s]).wait()
        base = cid * _CHUNK2
        pltpu.async_copy(stats_hbm.at[idxbuf.at[si]], gbuf.at[s], gsems[s])
        pltpu.async_copy(t_hbm.at[pl.ds(base, _CHUNK2)], xbuf.at[s],
                         xsems[s])

    def finish(k, u):
      s, si = u % 2, u % 4
      cid = wid + k * _NW

      @pl.when(jnp.logical_and(cid >= 0, cid < total_chunks))
      def _():
        base = cid * _CHUNK2
        pltpu.make_async_copy(
            stats_hbm.at[idxbuf.at[si]], gbuf.at[s], gsems[s]).wait()
        pltpu.make_async_copy(
            t_hbm.at[pl.ds(0, _CHUNK2)], xbuf.at[s], xsems[s]).wait()

        def grp_body(g, c):
          r0 = g * 16
          vseg = idxbuf[si, pl.ds(r0, 16)]
          uniform = vseg[0] == vseg[15]

          @pl.when(uniform)
          def _():
            mid = [gbuf[s, r0, pl.ds(_LANES * v, _LANES)] for v in range(nv)]
            inv = [gbuf[s, r0, pl.ds(d + _LANES * v, _LANES)]
                   for v in range(nv)]
            for j in range(16):
              for v in range(nv):
                sl = pl.ds(_LANES * v, _LANES)
                xbuf[s, r0 + j, sl] = (xbuf[s, r0 + j, sl] - mid[v]) * inv[v]

          @pl.when(jnp.logical_not(uniform))
          def _():
            def row_body(j, c2):
              for v in range(nv):
                sl = pl.ds(_LANES * v, _LANES)
                x = xbuf[s, j, sl]
                mid = gbuf[s, j, sl]
                inv = gbuf[s, j, pl.ds(d + _LANES * v, _LANES)]
                xbuf[s, j, sl] = (x - mid) * inv
              return c2

            lax.fori_loop(r0, r0 + 16, row_body, 0)

          return c

        lax.fori_loop(0, _CHUNK2 // 16, grp_body, 0)
        pltpu.async_copy(xbuf.at[s], out_hbm.at[pl.ds(base, _CHUNK2)],
                         osems[s])

      return

    n4 = (iters + 3) // 4
    prefetch_idx(0, 0)
    prefetch_idx(1, 1)
    prefetch_idx(2, 2)

    def loop4(k4, c):
      k = 4 * k4
      for u in range(4):
        start(k + u, u)
        finish(k + u - 1, u + 3)
        prefetch_idx(k + u + 3, (u + 3) % 4)
      return c

    lax.fori_loop(0, n4, loop4, 0)
    finish(4 * n4 - 1, 3)

    for s in range(2):
      pltpu.make_async_copy(
          xbuf.at[s], out_hbm.at[pl.ds(0, _CHUNK2)], osems[s]).wait()

  return pl.kernel(
      body,
      out_type=jax.ShapeDtypeStruct((n, d), jnp.float32),
      mesh=_make_mesh(),
      scratch_types=[
          pltpu.VMEM((2, _CHUNK2, d), jnp.float32),
          pltpu.VMEM((2, _CHUNK2, 2 * d), jnp.float32),
          pltpu.VMEM((4, _CHUNK2), jnp.int32),
          pltpu.SemaphoreType.DMA,
          pltpu.SemaphoreType.DMA,
          pltpu.SemaphoreType.DMA,
          pltpu.SemaphoreType.DMA,
          pltpu.SemaphoreType.DMA,
          pltpu.SemaphoreType.DMA,
          pltpu.SemaphoreType.DMA,
          pltpu.SemaphoreType.DMA,
          pltpu.SemaphoreType.DMA,
          pltpu.SemaphoreType.DMA,
      ],
      compiler_params=pltpu.CompilerParams(use_tc_tiling_on_sc=False),
      interpret=interpret,
  )


@jax.jit
def _run(tensor, segment_ids):
  seg16 = segment_ids.reshape(_N // 16, 16)
  segpad = jnp.concatenate([
      jnp.full((8,), -1, jnp.int32), segment_ids,
      jnp.full((16,), -1, jnp.int32)])
  stats = _build_phase1(_N, _D, _S)(tensor, seg16, segpad)
  return _build_phase2(_N, _D, _S)(tensor, segment_ids, stats)


def kernel(tensor, segment_ids, weight, bias, mean_scale):
  del weight, bias, mean_scale
  return _run(tensor, segment_ids)


# speedup vs baseline: 1.4133x; 1.4133x over previous
import functools

import jax
import jax.numpy as jnp
from jax import lax
from jax.experimental import pallas as pl
from jax.experimental.pallas import tpu as pltpu
from jax.experimental.pallas import tpu_sc as plsc

_N = 320000
_D = 128
_S = 10000
_NC = 2
_NS = 16
_LANES = 16
_NW = _NC * _NS

_CHUNK1 = 400
_CHUNK2 = 128

_EPS = 1e-12


def _make_mesh():
  return plsc.VectorSubcoreMesh(
      core_axis_name="c", subcore_axis_name="s",
      num_cores=_NC, num_subcores=_NS)


def _build_phase1(n, d, s_count, interpret=False):
  nv = d // _LANES
  p = n // _NW
  gpc = _CHUNK1 // 16
  nchunks = p // _CHUNK1
  gpw = p // 16
  total_chunks_all = n // _CHUNK1

  fb = 48
  fb_fire = 32

  def body(t_hbm, seg16_hbm, segp_hbm, stats_hbm, tbuf, segv, fsegv, pgbuf,
           flushbuf, flushidx, accbuf, smem, csem0, csem1, bsem0, bsem1,
           ext_sem):
    wid = lax.axis_index("s") * _NC + lax.axis_index("c")
    g0 = wid * gpw
    csems = (csem0, csem1)
    bsems = (bsem0, bsem1)
    lane0 = jnp.arange(16, dtype=jnp.int32) == 0
    pow2 = jnp.left_shift(jnp.int32(1), jnp.arange(16, dtype=jnp.int32))
    dummy_row = jnp.full((16,), jnp.int32(s_count))

    def reset_idx(sp):
      for i in range(fb // 16):
        flushidx[sp, pl.ds(i * 16, 16)] = dummy_row

    def flush(seg_splat, mx, mn):
      fc = smem[3]
      par = smem[4]
      for v in range(nv):
        mid = (mx[v] + mn[v]) * 0.5
        ldv = (mx[v] - mn[v]) * 0.5
        inv = 1.0 / jnp.maximum(ldv, _EPS)
        flushbuf[par, fc, pl.ds(_LANES * v, _LANES)] = mid
        flushbuf[par, fc, pl.ds(d + _LANES * v, _LANES)] = inv
      plsc.store_scatter(flushidx.at[par], [jnp.full((16,), fc)],
                         seg_splat, mask=lane0)
      smem[3] = fc + 1

    def fire_batch():
      par = smem[4]
      bfired = smem[5]
      for sp in range(2):
        @pl.when(par == sp)
        def _():
          pltpu.async_copy(
              flushbuf.at[sp], stats_hbm.at[flushidx.at[sp]], bsems[sp])

          @pl.when(bfired >= 1)
          def _():
            pltpu.make_async_copy(
                flushbuf.at[1 - sp], stats_hbm.at[flushidx.at[1 - sp]],
                bsems[1 - sp]).wait()
          reset_idx(1 - sp)

      smem[4] = 1 - par
      smem[3] = jnp.int32(0)
      smem[5] = bfired + 1

    def load_row(buf, slot, j):
      return [buf[slot, j, pl.ds(_LANES * v, _LANES)] for v in range(nv)]

    def load_acc():
      return ([accbuf[0, pl.ds(_LANES * v, _LANES)] for v in range(nv)],
              [accbuf[1, pl.ds(_LANES * v, _LANES)] for v in range(nv)])

    def store_acc(mx, mn):
      for v in range(nv):
        accbuf[0, pl.ds(_LANES * v, _LANES)] = mx[v]
        accbuf[1, pl.ds(_LANES * v, _LANES)] = mn[v]

    pltpu.sync_copy(seg16_hbm.at[jnp.maximum(g0 - 1, 0)], pgbuf)
    pgv = pgbuf[...]
    smem[0] = jnp.where(wid == 0, jnp.int32(-1), pgv[15])
    smem[1] = jnp.int32(0)
    smem[3] = jnp.int32(0)
    smem[4] = jnp.int32(0)
    smem[5] = jnp.int32(0)
    reset_idx(0)
    reset_idx(1)

    def issue_chunk(k, s):
      row0 = wid * p + k * _CHUNK1
      pltpu.async_copy(t_hbm.at[pl.ds(row0, _CHUNK1)], tbuf.at[s], csems[s])
      pltpu.async_copy(
          seg16_hbm.at[pl.ds(g0 + k * gpc, gpc)], segv.at[s], csems[s])
      pltpu.async_copy(
          segp_hbm.at[pl.ds(row0, _CHUNK1 + 16)], fsegv.at[s], csems[s])

    def wait_chunk(s):
      pltpu.make_async_copy(
          t_hbm.at[pl.ds(0, _CHUNK1)], tbuf.at[s], csems[s]).wait()
      pltpu.make_async_copy(
          seg16_hbm.at[pl.ds(0, gpc)], segv.at[s], csems[s]).wait()
      pltpu.make_async_copy(
          segp_hbm.at[pl.ds(0, _CHUNK1 + 16)], fsegv.at[s], csems[s]).wait()

    issue_chunk(0, 0)

    def chunk_body(k, c):
      slot = lax.rem(k, 2)

      @pl.when(slot == 0)
      def _():
        wait_chunk(0)

        @pl.when(k + 1 < nchunks)
        def _():
          issue_chunk(k + 1, 1)

      @pl.when(slot == 1)
      def _():
        wait_chunk(1)

        @pl.when(k + 1 < nchunks)
        def _():
          issue_chunk(k + 1, 0)

      def group_body(g, c2):
        prev = smem[0]
        vseg = segv[slot, g]
        last = vseg[15]
        fast_p = last == prev

        @pl.when(fast_p)
        def _():
          mx, mn = load_acc()
          for j in range(16):
            row = load_row(tbuf, slot, g * 16 + j)
            mx = [jnp.maximum(mx[v], row[v]) for v in range(nv)]
            mn = [jnp.minimum(mn[v], row[v]) for v in range(nv)]
          store_acc(mx, mn)

        @pl.when(jnp.logical_not(fast_p))
        def _():
          started0 = smem[1]
          mx, mn = load_acc()
          shifted = fsegv[slot, pl.ds(g * 16 + 7, 16)]
          bvec = vseg != shifted
          mbits = jnp.sum(jnp.where(bvec, pow2, 0))
          prev_splat = jnp.full((16,), prev)

          def slow_row(j, c3):
            started = c3[0]
            mx, mn = list(c3[1:1 + nv]), list(c3[1 + nv:])
            row = load_row(tbuf, slot, g * 16 + j)
            bj = jnp.bitwise_and(jnp.right_shift(mbits, j), 1)
            boundary = bj != 0
            do_flush = jnp.logical_and(boundary, started != 0)

            @pl.when(do_flush)
            def _():
              seg_splat = jnp.where(
                  j == 0, prev_splat,
                  jnp.take_along_axis(
                      vseg,
                      jnp.full((16,), jnp.maximum(j - 1, 0), jnp.int32),
                      axis=0))
              flush(seg_splat, mx, mn)

            mx = [jnp.where(boundary, row[v], jnp.maximum(mx[v], row[v]))
                  for v in range(nv)]
            mn = [jnp.where(boundary, row[v], jnp.minimum(mn[v], row[v]))
                  for v in range(nv)]
            started = jnp.bitwise_or(started, bj)
            return (started,) + tuple(mx) + tuple(mn)

          out = lax.fori_loop(0, 16, slow_row,
                              (started0,) + tuple(mx) + tuple(mn))
          store_acc(list(out[1:1 + nv]), list(out[1 + nv:]))
          smem[0] = last
          smem[1] = out[0]

          @pl.when(smem[3] >= fb_fire)
          def _():
            fire_batch()

        return c2

      return lax.fori_loop(0, gpc, group_body, c)

    lax.fori_loop(0, nchunks, chunk_body, 0)
    prev = smem[0]
    started = smem[1] != 0

    smem[2] = jnp.where(started, jnp.int32(0), jnp.int32(1))

    def ext_chunk(e, c):
      cc = (wid + 1) * nchunks + e

      @pl.when(jnp.logical_and(smem[2] == 0, cc < total_chunks_all))
      def _():
        pltpu.async_copy(
            t_hbm.at[pl.ds(cc * _CHUNK1, _CHUNK1)], tbuf.at[0],
            ext_sem).wait()
        pltpu.async_copy(
            seg16_hbm.at[pl.ds(cc * gpc, gpc)], segv.at[0], ext_sem).wait()

        def ext_group(g, c2):
          @pl.when(smem[2] == 0)
          def _():
            vseg = segv[0, g]
            mx, mn = load_acc()
            mb = jnp.sum(jnp.where(vseg != jnp.full((16,), prev), pow2, 0))

            def ext_row(j, c3):
              mx = list(c3[:nv])
              mn = list(c3[nv:])
              row = load_row(tbuf, 0, g * 16 + j)
              keep = jnp.bitwise_and(
                  mb, jnp.left_shift(jnp.int32(2), j) - 1) == 0
              mx = [jnp.where(keep, jnp.maximum(mx[v], row[v]), mx[v])
                    for v in range(nv)]
              mn = [jnp.where(keep, jnp.minimum(mn[v], row[v]), mn[v])
                    for v in range(nv)]
              return tuple(mx) + tuple(mn)

            out = lax.fori_loop(0, 16, ext_row, tuple(mx) + tuple(mn))
            store_acc(list(out[:nv]), list(out[nv:]))
            smem[2] = jnp.where(mb == 0, jnp.int32(0), jnp.int32(1))

          return c2

        lax.fori_loop(0, gpc, ext_group, 0)

      return c

    lax.fori_loop(0, total_chunks_all - nchunks, ext_chunk, 0)

    @pl.when(started)
    def _():
      mx, mn = load_acc()
      flush(jnp.full((16,), prev), mx, mn)

    @pl.when(smem[3] > 0)
    def _():
      fire_batch()

    @pl.when(smem[5] >= 1)
    def _():
      last = 1 - smem[4]
      for sp in range(2):
        @pl.when(last == sp)
        def _():
          pltpu.make_async_copy(
              flushbuf.at[sp], stats_hbm.at[flushidx.at[sp]],
              bsems[sp]).wait()

  return pl.kernel(
      body,
      out_type=jax.ShapeDtypeStruct((s_count + 8, 2 * d), jnp.float32),
      mesh=_make_mesh(),
      scratch_types=[
          pltpu.VMEM((2, _CHUNK1, d), jnp.float32),
          pltpu.VMEM((2, gpc, 16), jnp.int32),
          pltpu.VMEM((2, _CHUNK1 + 16), jnp.int32),
          pltpu.VMEM((16,), jnp.int32),
          pltpu.VMEM((2, fb, 2 * d), jnp.float32),
          pltpu.VMEM((2, fb), jnp.int32),
          pltpu.VMEM((2, d), jnp.float32),
          pltpu.SMEM((8,), jnp.int32),
          pltpu.SemaphoreType.DMA,
          pltpu.SemaphoreType.DMA,
          pltpu.SemaphoreType.DMA,
          pltpu.SemaphoreType.DMA,
          pltpu.SemaphoreType.DMA,
      ],
      compiler_params=pltpu.CompilerParams(
          use_tc_tiling_on_sc=False, needs_layout_passes=False),
      interpret=interpret,
  )


def _build_phase2(n, d, s_count, interpret=False):
  nv = d // _LANES
  total_chunks = n // _CHUNK2
  iters = -(-total_chunks // _NW)

  ngrp = _CHUNK2 // 8

  def body(t_hbm, segf_hbm, stats_hbm, out_hbm, xbuf, ghbuf, ggbuf, idxbuf,
           hbuf, smem, isem0, isem1, isem2, isem3, gsem0, gsem1,
           xsem0, xsem1, osem0, osem1):
    wid = lax.axis_index("s") * _NC + lax.axis_index("c")
    isems = (isem0, isem1, isem2, isem3)
    gsems = (gsem0, gsem1)
    xsems = (xsem0, xsem1)
    osems = (osem0, osem1)
    pow2 = jnp.left_shift(jnp.int32(1), jnp.arange(16, dtype=jnp.int32))
    head_il = jnp.arange(16, dtype=jnp.int32) * 8
    tail_il = head_il + 7

    def prefetch_idx(k, si):
      cid = wid + k * _NW

      @pl.when(jnp.logical_and(cid >= 0, cid < total_chunks))
      def _():
        pltpu.async_copy(
            segf_hbm.at[pl.ds(cid * _CHUNK2, _CHUNK2)], idxbuf.at[si],
            isems[si])

    def start(k, u):
      s, si = u % 2, u % 4
      cid = wid + k * _NW
      prev_cid = cid - 2 * _NW

      @pl.when(jnp.logical_and(cid >= 0, cid < total_chunks))
      def _():
        pltpu.make_async_copy(
            segf_hbm.at[pl.ds(0, _CHUNK2)], idxbuf.at[si], isems[si]).wait()

        @pl.when(prev_cid >= 0)
        def _():
          pltpu.make_async_copy(
              xbuf.at[s], out_hbm.at[pl.ds(0, _CHUNK2)], osems[s]).wait()
        base = cid * _CHUNK2
        heads = plsc.load_gather(idxbuf.at[si], [head_il])
        tails = plsc.load_gather(idxbuf.at[si], [tail_il])
        hbuf[s, :] = heads
        nub = jnp.sum(jnp.where(heads != tails, pow2, 0))
        smem[s] = nub
        pltpu.async_copy(stats_hbm.at[hbuf.at[s]], ghbuf.at[s], gsems[s])
        pltpu.async_copy(t_hbm.at[pl.ds(base, _CHUNK2)], xbuf.at[s],
                         xsems[s])
        for g in range(ngrp):
          @pl.when(jnp.bitwise_and(jnp.right_shift(nub, g), 1) != 0)
          def _():
            pltpu.async_copy(
                stats_hbm.at[idxbuf.at[si, pl.ds(g * 8, 8)]],
                ggbuf.at[s, g], gsems[s])

    def finish(k, u):
      s, si = u % 2, u % 4
      cid = wid + k * _NW

      @pl.when(jnp.logical_and(cid >= 0, cid < total_chunks))
      def _():
        base = cid * _CHUNK2
        nub = smem[s]
        pltpu.make_async_copy(
            stats_hbm.at[hbuf.at[s]], ghbuf.at[s], gsems[s]).wait()
        for g in range(ngrp):
          @pl.when(jnp.bitwise_and(jnp.right_shift(nub, g), 1) != 0)
          def _():
            pltpu.make_async_copy(
                stats_hbm.at[idxbuf.at[si, pl.ds(g * 8, 8)]],
                ggbuf.at[s, g], gsems[s]).wait()
        pltpu.make_async_copy(
            t_hbm.at[pl.ds(0, _CHUNK2)], xbuf.at[s], xsems[s]).wait()

        def grp_body(g, c):
          r0 = g * 8
          uniform = jnp.bitwise_and(jnp.right_shift(nub, g), 1) == 0

          @pl.when(uniform)
          def _():
            mid = [ghbuf[s, g, pl.ds(_LANES * v, _LANES)] for v in range(nv)]
            inv = [ghbuf[s, g, pl.ds(d + _LANES * v, _LANES)]
                   for v in range(nv)]
            for j in range(8):
              for v in range(nv):
                sl = pl.ds(_LANES * v, _LANES)
                xbuf[s, r0 + j, sl] = (xbuf[s, r0 + j, sl] - mid[v]) * inv[v]

          @pl.when(jnp.logical_not(uniform))
          def _():
            def row_body(j, c2):
              for v in range(nv):
                sl = pl.ds(_LANES * v, _LANES)
                x = xbuf[s, r0 + j, sl]
                mid = ggbuf[s, g, j, sl]
                inv = ggbuf[s, g, j, pl.ds(d + _LANES * v, _LANES)]
                xbuf[s, r0 + j, sl] = (x - mid) * inv
              return c2

            lax.fori_loop(0, 8, row_body, 0)

          return c

        lax.fori_loop(0, ngrp, grp_body, 0)
        pltpu.async_copy(xbuf.at[s], out_hbm.at[pl.ds(base, _CHUNK2)],
                         osems[s])

      return

    n4 = (iters + 3) // 4
    prefetch_idx(0, 0)
    prefetch_idx(1, 1)
    prefetch_idx(2, 2)

    def loop4(k4, c):
      k = 4 * k4
      for u in range(4):
        start(k + u, u)
        finish(k + u - 1, u + 3)
        prefetch_idx(k + u + 3, (u + 3) % 4)
      return c

    lax.fori_loop(0, n4, loop4, 0)
    finish(4 * n4 - 1, 3)

    for s in range(2):
      pltpu.make_async_copy(
          xbuf.at[s], out_hbm.at[pl.ds(0, _CHUNK2)], osems[s]).wait()

  return pl.kernel(
      body,
      out_type=jax.ShapeDtypeStruct((n, d), jnp.float32),
      mesh=_make_mesh(),
      scratch_types=[
          pltpu.VMEM((2, _CHUNK2, d), jnp.float32),
          pltpu.VMEM((2, 16, 2 * d), jnp.float32),
          pltpu.VMEM((2, ngrp, 8, 2 * d), jnp.float32),
          pltpu.VMEM((4, _CHUNK2), jnp.int32),
          pltpu.VMEM((2, 16), jnp.int32),
          pltpu.SMEM((2,), jnp.int32),
          pltpu.SemaphoreType.DMA,
          pltpu.SemaphoreType.DMA,
          pltpu.SemaphoreType.DMA,
          pltpu.SemaphoreType.DMA,
          pltpu.SemaphoreType.DMA,
          pltpu.SemaphoreType.DMA,
          pltpu.SemaphoreType.DMA,
          pltpu.SemaphoreType.DMA,
          pltpu.SemaphoreType.DMA,
          pltpu.SemaphoreType.DMA,
      ],
      compiler_params=pltpu.CompilerParams(
          use_tc_tiling_on_sc=False, needs_layout_passes=False),
      interpret=interpret,
  )


@jax.jit
def _run(tensor, segment_ids):
  seg16 = segment_ids.reshape(_N // 16, 16)
  segpad = jnp.concatenate([
      jnp.full((8,), -1, jnp.int32), segment_ids,
      jnp.full((16,), -1, jnp.int32)])
  stats = _build_phase1(_N, _D, _S)(tensor, seg16, segpad)
  return _build_phase2(_N, _D, _S)(tensor, segment_ids, stats)


def kernel(tensor, segment_ids, weight, bias, mean_scale):
  del weight, bias, mean_scale
  return _run(tensor, segment_ids)
